# baseline (speedup 1.0000x reference)
import functools
import jax
import jax.numpy as jnp
from jax import lax
from jax.experimental import pallas as pl
from jax.experimental.pallas import tpu as pltpu
from jax.experimental.pallas import tpu_sc as plsc

N = 100000
E = 3200000
NPAD = 100096
EPAD = 3244032
ROWS = EPAD // 128
NC = 2
NS = 16
CH = 3
CB = CH * 128
NB = 4
RPT = NPAD // NS

_MESH = plsc.VectorSubcoreMesh(core_axis_name="c", subcore_axis_name="s")
_SC_PARAMS = pltpu.CompilerParams(use_tc_tiling_on_sc=False)
_f32 = jnp.float32


def _make_edge_kernel(edge_split, gather, width):
    ntiles = NC * NS if edge_split else NS
    tile_subs = ROWS // CH // ntiles
    T = tile_subs
    G = T // NB

    scratch = [
        pltpu.VMEM((NB, 2, CB), jnp.int32),
        pltpu.VMEM((NB, CB, width), _f32),
        pltpu.VMEM_SHARED((NPAD, width), _f32),
        pltpu.SemaphoreType.DMA,
        pltpu.SemaphoreType.DMA,
        pltpu.SemaphoreType.DMA,
    ]

    @functools.partial(
        pl.kernel,
        out_type=jax.ShapeDtypeStruct((NC, NPAD, width), _f32),
        mesh=_MESH,
        compiler_params=_SC_PARAMS,
        scratch_types=scratch,
    )
    def edge_kernel(sd_hbm, gtab_hbm, zeros_hbm, out_hbm,
                    sdi, gbuf, acc, isem, gsem, ssem):
        c = lax.axis_index("c")
        s = lax.axis_index("s")
        pltpu.sync_copy(zeros_hbm.at[pl.ds(s * RPT, RPT)],
                        acc.at[pl.ds(s * RPT, RPT)])
        if not gather:
            pltpu.sync_copy(gtab_hbm.at[0], gbuf.at[0])
        plsc.subcore_barrier()
        if edge_split:
            base = (c * NS + s) * tile_subs
            tab = gtab_hbm.at[0]
        else:
            base = s * tile_subs
            tab = gtab_hbm.at[c]

        def idx_start(t, slot):
            pltpu.make_async_copy(sd_hbm.at[base + t], sdi.at[slot],
                                  isem).start()

        def idx_wait(slot):
            pltpu.make_async_copy(sd_hbm.at[base], sdi.at[slot], isem).wait()

        def gat_start(slot):
            pltpu.make_async_copy(tab.at[sdi.at[slot, 0]], gbuf.at[slot],
                                  gsem).start()

        def scat_pair(slot):
            sbuf = gbuf.at[slot] if gather else gbuf.at[0]
            return sbuf, acc.at[sdi.at[slot, 1]]

        def scat_start(slot):
            a, b2 = scat_pair(slot)
            pltpu.make_async_copy(a, b2, ssem).start(add=True)

        def scat_drain(slot):
            a, b2 = scat_pair(slot)
            pltpu.make_async_copy(a, b2, ssem).wait()

        idx_start(0, 0)
        idx_start(1, 1)
        idx_wait(0)
        if gather:
            gat_start(0)

        def step(g, carry):
            for b in range(NB):
                p = b
                q1 = (p + 1) % NB
                q2 = (p + 2) % NB
                t = g * NB + b

                if b >= NB - 2:
                    scat_drain(q2)
                else:
                    @pl.when(g >= 1)
                    def _():
                        scat_drain(q2)

                def head_bd():
                    idx_wait(q1)
                    if gather:
                        gat_start(q1)

                def issue_c():
                    idx_start(t + 2, q2)

                if b == NB - 1:
                    @pl.when(g < G - 1)
                    def _():
                        head_bd()
                        issue_c()
                else:
                    head_bd()
                    if b >= NB - 2:
                        @pl.when(g < G - 1)
                        def _():
                            issue_c()
                    else:
                        issue_c()

                if gather:
                    pltpu.make_async_copy(tab.at[sdi.at[p, 0]], gbuf.at[p],
                                          gsem).wait()
                scat_start(p)
            return carry

        lax.fori_loop(0, G, step, 0)
        for k in range(2):
            scat_drain((T - 2 + k) % NB)
        plsc.subcore_barrier()
        pltpu.sync_copy(acc.at[pl.ds(s * RPT, RPT)],
                        out_hbm.at[c].at[pl.ds(s * RPT, RPT)])

    return edge_kernel


_deg_call = _make_edge_kernel(edge_split=True, gather=False, width=8)
_l1_call = _make_edge_kernel(edge_split=False, gather=True, width=16)
_l2_call = _make_edge_kernel(edge_split=True, gather=True, width=16)


def _tc1_body(x_ref, w1_ref, dp_ref, gab_ref, dis_ref):
    deg = dp_ref[0, :, :1] + dp_ref[1, :, :1] + 1.0
    dis = lax.rsqrt(jnp.maximum(deg, 1e-12))
    h = jnp.dot(x_ref[...], w1_ref[...], preferred_element_type=_f32)
    g = h * dis
    gab_ref[0] = g[:, :16]
    gab_ref[1] = g[:, 16:]
    dis_ref[...] = dis


def _tc2_body(s1_ref, gab_ref, dis_ref, b1a_ref, b1b_ref, w2a_ref, w2b_ref,
              g2_ref):
    dis = dis_ref[...]
    h1a = jnp.maximum(dis * (s1_ref[0] + gab_ref[0]) + b1a_ref[...], 0.0)
    h1b = jnp.maximum(dis * (s1_ref[1] + gab_ref[1]) + b1b_ref[...], 0.0)
    h2 = (jnp.dot(h1a, w2a_ref[...], preferred_element_type=_f32)
          + jnp.dot(h1b, w2b_ref[...], preferred_element_type=_f32))
    g2_ref[...] = h2 * dis


def _tc3_body(s2_ref, g2_ref, dis_ref, b2_ref, z_ref):
    z_ref[...] = (dis_ref[...] * (s2_ref[0] + s2_ref[1] + g2_ref[...])
                  + b2_ref[...])


_B = 3128
_GRID = NPAD // _B


def _rows3(i):
    return (0, i, 0)


def _rows2(i):
    return (i, 0)


def _full2(i):
    return (0, 0)


_blk3 = pl.BlockSpec((NC, _B, 16), _rows3)
_blk2 = pl.BlockSpec((_B, 16), _rows2)
_blk1 = pl.BlockSpec((_B, 1), _rows2)

_tc1 = pl.pallas_call(
    _tc1_body,
    grid=(_GRID,),
    in_specs=[pl.BlockSpec((_B, 6), _rows2),
              pl.BlockSpec((6, 32), _full2),
              pl.BlockSpec((NC, _B, 8), _rows3)],
    out_specs=(_blk3, _blk1),
    out_shape=(jax.ShapeDtypeStruct((NC, NPAD, 16), _f32),
               jax.ShapeDtypeStruct((NPAD, 1), _f32)),
)
_tc2 = pl.pallas_call(
    _tc2_body,
    grid=(_GRID,),
    in_specs=[_blk3, _blk3, _blk1,
              pl.BlockSpec((1, 16), _full2), pl.BlockSpec((1, 16), _full2),
              pl.BlockSpec((16, 16), _full2), pl.BlockSpec((16, 16), _full2)],
    out_specs=_blk2,
    out_shape=jax.ShapeDtypeStruct((NPAD, 16), _f32),
)
_tc3 = pl.pallas_call(
    _tc3_body,
    grid=(_GRID,),
    in_specs=[_blk3, _blk2, _blk1, pl.BlockSpec((1, 16), _full2)],
    out_specs=_blk2,
    out_shape=jax.ShapeDtypeStruct((NPAD, 16), _f32),
)


@jax.jit
def kernel(x, edge_index, W1, b1, W2, b2):
    src = jnp.pad(edge_index[0], (0, EPAD - E), constant_values=N)
    dst = jnp.pad(edge_index[1], (0, EPAD - E), constant_values=N)
    sd = jnp.stack([src.reshape(-1, CB), dst.reshape(-1, CB)],
                   axis=1).astype(jnp.int32)
    xp = jnp.pad(x, ((0, NPAD - N), (0, 0)))
    zeros16 = jnp.zeros((NPAD, 16), _f32)
    zeros8 = jnp.zeros((NPAD, 8), _f32)
    ones = jnp.ones((1, CB, 8), _f32)

    degp = _deg_call(sd, ones, zeros8)
    gab, dis = _tc1(xp, W1, degp)
    s1 = _l1_call(sd, gab, zeros16)
    g2 = _tc2(s1, gab, dis, b1[:16][None, :], b1[16:][None, :],
              W2[:16], W2[16:])
    s2 = _l2_call(sd, g2[None], zeros16)
    z = _tc3(s2, g2, dis, b2[None, :])
    return z[:N]
